# Initial kernel scaffold; baseline (speedup 1.0000x reference)
import jax
import jax.numpy as jnp
from jax.experimental import pallas as pl


def kernel(indices, table):
    raise NotImplementedError("write your pallas kernel here")


# speedup vs baseline: 1.4771x; 1.4771x over previous
import functools

import jax
import jax.numpy as jnp
from jax import lax
from jax.experimental import pallas as pl
from jax.experimental.pallas import tpu as pltpu
from jax.experimental.pallas import tpu_sc as plsc


def _gather_kernel(n_total, n_per_w, chunk, num_cores, embed):
    n_chunks = n_per_w // chunk
    mesh = plsc.VectorSubcoreMesh(core_axis_name="c", subcore_axis_name="s")

    @functools.partial(
        pl.kernel,
        mesh=mesh,
        out_type=jax.ShapeDtypeStruct((n_total, embed), jnp.float32),
        scratch_types=[
            pltpu.VMEM((chunk,), jnp.int32),
            pltpu.VMEM((chunk, embed), jnp.float32),
            pltpu.SemaphoreType.DMA,
        ],
        compiler_params=pltpu.CompilerParams(use_tc_tiling_on_sc=False),
    )
    def k(idx_hbm, table_hbm, out_hbm, idx_v, rows_v, sem):
        wid = lax.axis_index("s") * num_cores + lax.axis_index("c")
        base = wid * n_per_w

        def body(i, carry):
            off = base + i * chunk
            pltpu.sync_copy(idx_hbm.at[pl.ds(off, chunk)], idx_v)
            pltpu.async_copy(table_hbm.at[idx_v], rows_v, sem).wait()
            pltpu.sync_copy(rows_v, out_hbm.at[pl.ds(off, chunk)])
            return carry

        lax.fori_loop(0, n_chunks, body, 0)

    return k


def kernel(indices, table):
    b, l = indices.shape
    v, embed = table.shape
    n_total = b * l
    info = plsc.get_sparse_core_info()
    nw = info.num_cores * info.num_subcores
    n_per_w = n_total // nw
    chunk = 1600
    k = _gather_kernel(n_total, n_per_w, chunk, info.num_cores, embed)
    out_flat = k(indices.reshape(n_total), table)
    return out_flat.reshape(b, l, embed)


# speedup vs baseline: 1.5004x; 1.0157x over previous
import functools

import jax
import jax.numpy as jnp
from jax import lax
from jax.experimental import pallas as pl
from jax.experimental.pallas import tpu as pltpu
from jax.experimental.pallas import tpu_sc as plsc

_NBUF = 2


def _gather_kernel(n_total, n_per_w, chunk, num_cores, embed):
    n_chunks = n_per_w // chunk
    mesh = plsc.VectorSubcoreMesh(core_axis_name="c", subcore_axis_name="s")

    scratch = (
        [pltpu.VMEM((chunk,), jnp.int32) for _ in range(_NBUF)]
        + [pltpu.VMEM((chunk, embed), jnp.float32) for _ in range(_NBUF)]
        + [pltpu.SemaphoreType.DMA for _ in range(2 * _NBUF)]
    )

    @functools.partial(
        pl.kernel,
        mesh=mesh,
        out_type=jax.ShapeDtypeStruct((n_total, embed), jnp.float32),
        scratch_types=scratch,
        compiler_params=pltpu.CompilerParams(use_tc_tiling_on_sc=False),
    )
    def k(idx_hbm, table_hbm, out_hbm, *bufs):
        idx_v = bufs[:_NBUF]
        rows_v = bufs[_NBUF:2 * _NBUF]
        gsem = bufs[2 * _NBUF:3 * _NBUF]
        osem = bufs[3 * _NBUF:]
        wid = lax.axis_index("s") * num_cores + lax.axis_index("c")
        base = wid * n_per_w

        pending_gather = [None] * _NBUF
        pending_out = [None] * _NBUF

        for b in range(min(_NBUF, n_chunks)):
            off = base + b * chunk
            pltpu.sync_copy(idx_hbm.at[pl.ds(off, chunk)], idx_v[b])
            pending_gather[b] = pltpu.async_copy(
                table_hbm.at[idx_v[b]], rows_v[b], gsem[b])

        for g in range(n_chunks):
            b = g % _NBUF
            off = base + g * chunk
            pending_gather[b].wait()
            pending_out[b] = pltpu.async_copy(
                rows_v[b], out_hbm.at[pl.ds(off, chunk)], osem[b])
            nxt = g + _NBUF
            if nxt < n_chunks:
                noff = base + nxt * chunk
                pltpu.sync_copy(idx_hbm.at[pl.ds(noff, chunk)], idx_v[b])
                pending_out[b].wait()
                pending_out[b] = None
                pending_gather[b] = pltpu.async_copy(
                    table_hbm.at[idx_v[b]], rows_v[b], gsem[b])

        for b in range(_NBUF):
            if pending_out[b] is not None:
                pending_out[b].wait()

    return k


def kernel(indices, table):
    b, l = indices.shape
    v, embed = table.shape
    n_total = b * l
    info = plsc.get_sparse_core_info()
    nw = info.num_cores * info.num_subcores
    n_per_w = n_total // nw
    chunk = 1600
    k = _gather_kernel(n_total, n_per_w, chunk, info.num_cores, embed)
    out_flat = k(indices.reshape(n_total), table)
    return out_flat.reshape(b, l, embed)
